# baseline (speedup 1.0000x reference)
import functools

import jax
import jax.numpy as jnp
from jax import lax
from jax.experimental import pallas as pl
from jax.experimental.pallas import tpu as pltpu
from jax.experimental.pallas import tpu_sc as plsc

NUM_TEAMS = 100000
EMBED_DIM = 128
INPUT_DIM = 256
HIDDEN_DIM = 1024
OUTPUT_DIM = 64
BATCH = 16384

NC = 2
NS = 16
NW = NC * NS
B_PER_W = BATCH // NW
CHUNK = 128
N_CHUNKS = B_PER_W // CHUNK


def _sc_gather_body(home_hbm, away_hbm, idx_h_hbm, idx_a_hbm,
                    home_out, away_out, idx_v, rows_v, sem):
  wid = lax.axis_index("s") * NC + lax.axis_index("c")
  base = wid * B_PER_W

  def one_table(table_hbm, idx_hbm, out_hbm):
    pltpu.sync_copy(idx_hbm.at[pl.ds(base, B_PER_W)], idx_v)
    copies = []
    for j in range(N_CHUNKS):
      copies.append(pltpu.async_copy(
          table_hbm.at[idx_v.at[pl.ds(j * CHUNK, CHUNK)]],
          rows_v.at[pl.ds(j * CHUNK, CHUNK)], sem))
    for c in copies:
      c.wait()
    pltpu.sync_copy(rows_v, out_hbm.at[pl.ds(base, B_PER_W)])

  one_table(home_hbm, idx_h_hbm, home_out)
  one_table(away_hbm, idx_a_hbm, away_out)


@functools.cache
def _get_sc_gather():
  return pl.kernel(
      _sc_gather_body,
      out_type=(
          jax.ShapeDtypeStruct((BATCH, EMBED_DIM), jnp.float32),
          jax.ShapeDtypeStruct((BATCH, EMBED_DIM), jnp.float32),
      ),
      mesh=plsc.VectorSubcoreMesh(core_axis_name="c", subcore_axis_name="s"),
      scratch_types=[
          pltpu.VMEM((B_PER_W,), jnp.int32),
          pltpu.VMEM((B_PER_W, EMBED_DIM), jnp.float32),
          pltpu.SemaphoreType.DMA,
      ],
  )


BM = 1024


def _mlp_body(home_ref, away_ref, xo_ref, w1_ref, b1_ref, w2_ref, b2_ref,
              out_ref):
  bf = jnp.bfloat16
  w1 = w1_ref[...].astype(bf)
  acc = jnp.dot(home_ref[...].astype(bf), w1[0:EMBED_DIM, :],
                preferred_element_type=jnp.float32)
  acc += jnp.dot(away_ref[...].astype(bf), w1[EMBED_DIM:2 * EMBED_DIM, :],
                 preferred_element_type=jnp.float32)
  acc += jnp.dot(xo_ref[...].astype(bf), w1[2 * EMBED_DIM:, :],
                 preferred_element_type=jnp.float32)
  h = jnp.maximum(acc + b1_ref[...], 0.0)
  out_ref[...] = jnp.dot(h.astype(bf), w2_ref[...].astype(bf),
                         preferred_element_type=jnp.float32) + b2_ref[...]


def _mlp(home, away, x_other, w1, b1, w2, b2):
  grid = (BATCH // BM,)
  return pl.pallas_call(
      _mlp_body,
      grid=grid,
      in_specs=[
          pl.BlockSpec((BM, EMBED_DIM), lambda i: (i, 0)),
          pl.BlockSpec((BM, EMBED_DIM), lambda i: (i, 0)),
          pl.BlockSpec((BM, INPUT_DIM), lambda i: (i, 0)),
          pl.BlockSpec((2 * EMBED_DIM + INPUT_DIM, HIDDEN_DIM),
                       lambda i: (0, 0)),
          pl.BlockSpec((1, HIDDEN_DIM), lambda i: (0, 0)),
          pl.BlockSpec((HIDDEN_DIM, OUTPUT_DIM), lambda i: (0, 0)),
          pl.BlockSpec((1, OUTPUT_DIM), lambda i: (0, 0)),
      ],
      out_specs=pl.BlockSpec((BM, OUTPUT_DIM), lambda i: (i, 0)),
      out_shape=jax.ShapeDtypeStruct((BATCH, OUTPUT_DIM), jnp.float32),
      compiler_params=pltpu.CompilerParams(
          dimension_semantics=("arbitrary",),
      ),
  )(home, away, x_other, w1, b1, w2, b2)


@jax.jit
def kernel(x_teams, x_other, home_table, away_table, W1, b1, W2, b2):
  idx_home = x_teams[:, 0]
  idx_away = x_teams[:, 1]
  home_rows, away_rows = _get_sc_gather()(
      home_table, away_table, idx_home, idx_away)
  return _mlp(home_rows, away_rows, x_other, W1,
              b1.reshape(1, HIDDEN_DIM), W2, b2.reshape(1, OUTPUT_DIM))


# speedup vs baseline: 1.1501x; 1.1501x over previous
import functools

import jax
import jax.numpy as jnp
from jax import lax
from jax.experimental import pallas as pl
from jax.experimental.pallas import tpu as pltpu
from jax.experimental.pallas import tpu_sc as plsc

NUM_TEAMS = 100000
EMBED_DIM = 128
INPUT_DIM = 256
HIDDEN_DIM = 1024
OUTPUT_DIM = 64
BATCH = 16384

NC = 2
NS = 16
NW = NC * NS
B_PER_W = BATCH // NW
CHUNK = 128
N_CHUNKS = B_PER_W // CHUNK


def _sc_gather_body(home_hbm, away_hbm, idx_h_hbm, idx_a_hbm,
                    home_out, away_out, idx_v, rows_v, sem):
  wid = lax.axis_index("s") * NC + lax.axis_index("c")
  base = wid * B_PER_W

  def one_table(table_hbm, idx_hbm, out_hbm):
    pltpu.sync_copy(idx_hbm.at[pl.ds(base, B_PER_W)], idx_v)
    copies = []
    for j in range(N_CHUNKS):
      copies.append(pltpu.async_copy(
          table_hbm.at[idx_v.at[pl.ds(j * CHUNK, CHUNK)]],
          rows_v.at[pl.ds(j * CHUNK, CHUNK)], sem))
    for c in copies:
      c.wait()
    pltpu.sync_copy(rows_v, out_hbm.at[pl.ds(base, B_PER_W)])

  one_table(home_hbm, idx_h_hbm, home_out)
  one_table(away_hbm, idx_a_hbm, away_out)


@functools.cache
def _get_sc_gather():
  return pl.kernel(
      _sc_gather_body,
      out_type=(
          jax.ShapeDtypeStruct((BATCH, EMBED_DIM), jnp.float32),
          jax.ShapeDtypeStruct((BATCH, EMBED_DIM), jnp.float32),
      ),
      mesh=plsc.VectorSubcoreMesh(core_axis_name="c", subcore_axis_name="s"),
      scratch_types=[
          pltpu.VMEM((B_PER_W,), jnp.int32),
          pltpu.VMEM((B_PER_W, EMBED_DIM), jnp.float32),
          pltpu.SemaphoreType.DMA,
      ],
  )


BM = 1024


def _mlp_body(home_ref, away_ref, xo_ref, w1_ref, b1_ref, w2_ref, b2_ref,
              out_ref):
  bf = jnp.bfloat16
  x = jnp.concatenate([home_ref[...].astype(bf), away_ref[...].astype(bf),
                       xo_ref[...].astype(bf)], axis=1)
  acc = jnp.dot(x, w1_ref[...], preferred_element_type=jnp.float32)
  h = jnp.maximum(acc + b1_ref[...], 0.0)
  out_ref[...] = jnp.dot(h.astype(bf), w2_ref[...],
                         preferred_element_type=jnp.float32) + b2_ref[...]


def _mlp(home, away, x_other, w1, b1, w2, b2):
  grid = (BATCH // BM,)
  return pl.pallas_call(
      _mlp_body,
      grid=grid,
      in_specs=[
          pl.BlockSpec((BM, EMBED_DIM), lambda i: (i, 0)),
          pl.BlockSpec((BM, EMBED_DIM), lambda i: (i, 0)),
          pl.BlockSpec((BM, INPUT_DIM), lambda i: (i, 0)),
          pl.BlockSpec((2 * EMBED_DIM + INPUT_DIM, HIDDEN_DIM),
                       lambda i: (0, 0)),
          pl.BlockSpec((1, HIDDEN_DIM), lambda i: (0, 0)),
          pl.BlockSpec((HIDDEN_DIM, OUTPUT_DIM), lambda i: (0, 0)),
          pl.BlockSpec((1, OUTPUT_DIM), lambda i: (0, 0)),
      ],
      out_specs=pl.BlockSpec((BM, OUTPUT_DIM), lambda i: (i, 0)),
      out_shape=jax.ShapeDtypeStruct((BATCH, OUTPUT_DIM), jnp.float32),
      compiler_params=pltpu.CompilerParams(
          dimension_semantics=("arbitrary",),
      ),
  )(home, away, x_other, w1, b1, w2, b2)


@jax.jit
def kernel(x_teams, x_other, home_table, away_table, W1, b1, W2, b2):
  idx_home = x_teams[:, 0]
  idx_away = x_teams[:, 1]
  home_rows, away_rows = _get_sc_gather()(
      home_table, away_table, idx_home, idx_away)
  return _mlp(home_rows, away_rows, x_other, W1.astype(jnp.bfloat16),
              b1.reshape(1, HIDDEN_DIM), W2.astype(jnp.bfloat16),
              b2.reshape(1, OUTPUT_DIM))
